# baseline (speedup 1.0000x reference)
import functools

import jax
import jax.numpy as jnp
import numpy as np
from jax import lax
from jax.experimental import pallas as pl
from jax.experimental.pallas import tpu as pltpu
from jax.experimental.pallas import tpu_sc as plsc

N_MODES = 16
PO2 = 2.0 * np.pi
PO4 = 4.0 * np.pi
INV_PI = 1.0 / np.pi
LN2 = 0.6931471805599453
PI_HI = np.float32(3.1415927410125732)
PI_LO = np.float32(-8.742277657347586e-08)

SC_CHUNKS = 16


def _sincos(theta):
    tq = theta * np.float32(INV_PI)
    u = tq + 0.5
    iu = u.astype(jnp.int32)
    fu = iu.astype(jnp.float32)
    ki = iu - jnp.where(fu > u, 1, 0)
    kf = ki.astype(jnp.float32)
    x = theta - kf * PI_HI
    x = x - kf * PI_LO
    x2 = x * x
    c = np.float32(1.0 / 479001600.0)
    for coef in (-1.0 / 3628800.0, 1.0 / 40320.0, -1.0 / 720.0,
                 1.0 / 24.0, -0.5, 1.0):
        c = c * x2 + np.float32(coef)
    s = np.float32(-1.0 / 39916800.0)
    for coef in (1.0 / 362880.0, -1.0 / 5040.0, 1.0 / 120.0,
                 -1.0 / 6.0, 1.0):
        s = s * x2 + np.float32(coef)
    s = s * x
    sgn = jnp.where((ki & 1) == 1, np.float32(-1.0), np.float32(1.0))
    return s * sgn, c * sgn


def _plog(s):
    ib = lax.bitcast_convert_type(s, jnp.int32)
    e = lax.shift_right_arithmetic(ib, 23) - 127
    mb = lax.bitwise_or(lax.bitwise_and(ib, 0x7FFFFF), 0x3F800000)
    m = lax.bitcast_convert_type(mb, jnp.float32)
    big = m > np.float32(1.4142135)
    m = jnp.where(big, m * 0.5, m)
    ef = e.astype(jnp.float32) + jnp.where(big, np.float32(1.0), np.float32(0.0))
    un = m - 1.0
    t = un / (un + 2.0)
    t2 = t * t
    p = np.float32(1.0 / 7.0)
    for coef in (1.0 / 5.0, 1.0 / 3.0, 1.0):
        p = p * t2 + np.float32(coef)
    return ef * np.float32(LN2) + 2.0 * t * p


def _sample_math(rdn, e0, e1, w0, w1, wlast, ss0, ss1, ls0, ls1,
                 lamb_in, lamb_out):
    mask = rdn < wlast
    cond1 = jnp.abs(w0) > jnp.abs(w1)
    zero_pos = jnp.logical_and(w0 == 0.0, w1 == 0.0)
    cond2 = jnp.logical_and(~cond1, ~zero_pos)
    d0 = jnp.where(w0 == 0.0, 1.0, w0)
    d1 = jnp.where(w1 == 0.0, 1.0, w1)
    num = jnp.where(cond1, w1, w0)
    den = jnp.where(cond1, d0, d1)
    t = np.float32(PO4) * num / den
    theta = jnp.where(cond1, t, np.float32(PO2) - t)
    r = jnp.where(cond1, w0, jnp.where(cond2, w1, np.float32(0.0)))
    sv, cv = _sincos(theta)
    z0 = jnp.where(mask, r * cv, e0 * ss0 + ls0)
    z1 = jnp.where(mask, r * sv, e1 * ss1 + ls1)
    z0s = z0 * z0
    z1s = z1 * z1
    acc = jnp.exp(jnp.where(z0s + z1s >= 1.0, lamb_out, lamb_in))
    return z0, z1, z0s, z1s, acc


def _pack_consts(loc, log_scale, weight_scores):
    w = jax.nn.softmax(weight_scores, axis=1)[0]
    wlast = w[-1]
    lc = loc[0]
    sc = jnp.exp(log_scale[0])
    b = 0.5 / (sc * sc)
    a = (-0.5 * 2.0 * np.log(2.0 * np.pi)
         + jnp.log(w[:-1] + 1e-05) - log_scale[0].sum(axis=1))
    cc = a - b[:, 0] * lc[:, 0] ** 2 - b[:, 1] * lc[:, 1] ** 2
    p0 = 2.0 * b[:, 0] * lc[:, 0]
    p1 = 2.0 * b[:, 1] * lc[:, 1]
    ss = sc.sum(axis=0)
    ls = lc.sum(axis=0)
    lamb_in = jnp.log(INV_PI + 1e-05) + jnp.log(wlast)
    lamb_out = jnp.log(2e-05) + jnp.log(wlast)
    m_stat = jnp.maximum(jnp.maximum(a.max(), lamb_in), lamb_out)
    head = jnp.stack([wlast, ss[0], ss[1], ls[0], ls[1],
                      lamb_in - m_stat, lamb_out - m_stat, m_stat])
    return jnp.concatenate([head, cc - m_stat, p0, p1, b[:, 0], b[:, 1]])


def _tc_body(consts_ref, rdn_ref, e0_ref, e1_ref, w0_ref, w1_ref,
             z0_ref, z1_ref, lp_ref):
    c = consts_ref
    z0, z1, z0s, z1s, acc = _sample_math(
        rdn_ref[...], e0_ref[...], e1_ref[...], w0_ref[...], w1_ref[...],
        c[0], c[1], c[2], c[3], c[4], c[5], c[6])
    for m in range(N_MODES):
        lp = (c[8 + m] + c[24 + m] * z0 + c[40 + m] * z1
              - c[56 + m] * z0s - c[72 + m] * z1s)
        acc = acc + jnp.exp(lp)
    z0_ref[...] = z0
    z1_ref[...] = z1
    lp_ref[...] = c[7] + jnp.log(acc)


def _tc_call(consts, rdn, e0, e1, w0, w1):
    n = rdn.shape[0]
    blk = min(n, 65536)
    grid = (n // blk,)
    vec = lambda: pl.BlockSpec((blk,), lambda i: (i,))
    return pl.pallas_call(
        _tc_body,
        grid=grid,
        in_specs=[pl.BlockSpec(memory_space=pltpu.SMEM)]
        + [vec() for _ in range(5)],
        out_specs=[vec() for _ in range(3)],
        out_shape=[jax.ShapeDtypeStruct((n,), jnp.float32) for _ in range(3)],
    )(consts, rdn, e0, e1, w0, w1)


_NW = 32
_CH = 2048
_VEC = 8


def _sc_row(cons_v, r):
    return cons_v[pl.ds(r * 16, 16)]


def _sc_body(consts_h, rdn_h, eps_h, wo_h, z_h, lp_h,
             cons_v, rdn_v, eps_v, wo_v, z_v, lp_v):
    wid = lax.axis_index("s") * 2 + lax.axis_index("c")
    shard = rdn_h.shape[0] // _NW
    pltpu.sync_copy(consts_h, cons_v)
    wlast = _sc_row(cons_v, 0)
    ss0 = _sc_row(cons_v, 1)
    ss1 = _sc_row(cons_v, 2)
    ls0 = _sc_row(cons_v, 3)
    ls1 = _sc_row(cons_v, 4)
    lamb_in = _sc_row(cons_v, 5)
    lamb_out = _sc_row(cons_v, 6)
    m_stat = _sc_row(cons_v, 7)
    iota2 = 2 * lax.iota(jnp.int32, 16)

    def chunk_body(ch, carry):
        base = wid * shard + ch * _CH
        pltpu.sync_copy(rdn_h.at[pl.ds(base, _CH)], rdn_v)
        pltpu.sync_copy(eps_h.at[pl.ds(2 * base, 2 * _CH)], eps_v)
        pltpu.sync_copy(wo_h.at[pl.ds(2 * base, 2 * _CH)], wo_v)

        def vec_body(i, carry2):
            offs = [i * (16 * _VEC) + v * 16 for v in range(_VEC)]
            zs = []
            for off in offs:
                idx0 = 2 * off + iota2
                idx1 = idx0 + 1
                e0 = plsc.load_gather(eps_v, [idx0])
                e1 = plsc.load_gather(eps_v, [idx1])
                w0 = plsc.load_gather(wo_v, [idx0])
                w1 = plsc.load_gather(wo_v, [idx1])
                rdn = rdn_v[pl.ds(off, 16)]
                zs.append(_sample_math(rdn, e0, e1, w0, w1, wlast, ss0, ss1,
                                       ls0, ls1, lamb_in, lamb_out))
            accs = [t[4] for t in zs]
            for m in range(N_MODES):
                cm = _sc_row(cons_v, 8 + m)
                p0m = _sc_row(cons_v, 24 + m)
                p1m = _sc_row(cons_v, 40 + m)
                q0m = _sc_row(cons_v, 56 + m)
                q1m = _sc_row(cons_v, 72 + m)
                for v in range(_VEC):
                    z0, z1, z0s, z1s, _ = zs[v]
                    lp = cm + p0m * z0 + p1m * z1 - q0m * z0s - q1m * z1s
                    accs[v] = accs[v] + jnp.exp(lp)
            for v, off in enumerate(offs):
                z0, z1, _, _, _ = zs[v]
                idx0 = 2 * off + iota2
                plsc.store_scatter(z_v, [idx0], z0)
                plsc.store_scatter(z_v, [idx0 + 1], z1)
                lp_v[pl.ds(off, 16)] = m_stat + _plog(accs[v])
            return carry2

        lax.fori_loop(0, _CH // (16 * _VEC), vec_body, 0)
        pltpu.sync_copy(z_v, z_h.at[pl.ds(2 * base, 2 * _CH)])
        pltpu.sync_copy(lp_v, lp_h.at[pl.ds(base, _CH)])
        return carry

    lax.fori_loop(0, shard // _CH, chunk_body, 0)


def _sc_call(consts_b, rdn, eps_flat, wo_flat):
    n = rdn.shape[0]
    run = pl.kernel(
        _sc_body,
        out_type=[jax.ShapeDtypeStruct((2 * n,), jnp.float32),
                  jax.ShapeDtypeStruct((n,), jnp.float32)],
        mesh=plsc.VectorSubcoreMesh(core_axis_name="c", subcore_axis_name="s"),
        compiler_params=pltpu.CompilerParams(needs_layout_passes=False),
        scratch_types=[pltpu.VMEM((88 * 16,), jnp.float32),
                       pltpu.VMEM((_CH,), jnp.float32),
                       pltpu.VMEM((2 * _CH,), jnp.float32),
                       pltpu.VMEM((2 * _CH,), jnp.float32),
                       pltpu.VMEM((2 * _CH,), jnp.float32),
                       pltpu.VMEM((_CH,), jnp.float32)],
    )
    return run(consts_b, rdn, eps_flat, wo_flat)


def kernel(num_samples, loc, log_scale, weight_scores, rdn, eps, wo):
    n = rdn.shape[0]
    consts = _pack_consts(loc, log_scale, weight_scores)
    consts_b = jnp.broadcast_to(consts[:, None], (88, 16)).reshape(-1)

    n_sc = min(n, SC_CHUNKS * _NW * _CH)
    z_parts, lp_parts = [], []
    if n_sc:
        zf, lp_sc = _sc_call(consts_b, rdn[:n_sc],
                             eps[:n_sc].reshape(-1), wo[:n_sc].reshape(-1))
        z_parts.append(zf.reshape(n_sc, 2))
        lp_parts.append(lp_sc)
    if n_sc < n:
        r = slice(n_sc, n)
        z0, z1, lp_tc = _tc_call(consts, rdn[r], eps[r, 0], eps[r, 1],
                                 wo[r, 0], wo[r, 1])
        z_parts.append(jnp.stack([z0, z1], axis=1))
        lp_parts.append(lp_tc)
    z = z_parts[0] if len(z_parts) == 1 else jnp.concatenate(z_parts)
    lp = lp_parts[0] if len(lp_parts) == 1 else jnp.concatenate(lp_parts)
    return z, lp
